# baseline (speedup 1.0000x reference)
import functools
import math

import jax
import jax.numpy as jnp
from jax import lax
from jax.experimental import pallas as pl
from jax.experimental.pallas import tpu as pltpu
from jax.experimental.pallas import tpu_sc as plsc

N = 10000
E = 320000
D = 128
H = 128
FT = 64
P = 20

NP = 10240

NC = 2
NS = 16
NW = NC * NS
C = 80
NCH = E // NW // C
EPW = NCH * C
RPS = NP // NS
G16 = C // 16
NSL = 5
NCS = NCH // NSL

RB = 320


def _mesh():
    return plsc.VectorSubcoreMesh(core_axis_name="c", subcore_axis_name="s")


def _wid():
    return lax.axis_index("s") * NC + lax.axis_index("c")


def _sc_deg(dst_rs, onerows, zeros128):
    @functools.partial(
        pl.kernel,
        out_type=jax.ShapeDtypeStruct((NC, NP, H), jnp.float32),
        mesh=_mesh(),
        scratch_types=[
            pltpu.VMEM((NSL, NCS, C), jnp.int32),
            pltpu.VMEM((C, H), jnp.float32),
            pltpu.VMEM_SHARED((NP, H), jnp.float32),
            pltpu.SemaphoreType.DMA,
            pltpu.SemaphoreType.DMA,
        ],
    )
    def k(dst_hbm, ones_hbm, z_hbm, out_hbm, didx, ones_v, acc, semA, semB):
        c = lax.axis_index("c")
        s = lax.axis_index("s")
        w = _wid()
        pltpu.sync_copy(z_hbm, acc.at[pl.ds(s * RPS, RPS)])
        pltpu.sync_copy(ones_hbm, ones_v)
        pltpu.sync_copy(dst_hbm.at[w], didx)
        plsc.subcore_barrier()

        def body(t, carry):
            ds_ = []
            for u in range(4):
                a = 4 * t + u
                ds_.append(pltpu.async_copy(
                    ones_v, acc.at[didx.at[a // NCS, a % NCS]],
                    semA if u % 2 == 0 else semB, add=True))
            for d in ds_:
                d.wait()
            return carry

        lax.fori_loop(0, NCH // 4, body, 0)
        pltpu.sync_copy(
            ones_v, acc.at[didx.at[NSL - 1, NCS - 1]], add=True)
        plsc.subcore_barrier()
        pltpu.sync_copy(acc.at[pl.ds(s * RPS, RPS)],
                        out_hbm.at[c, pl.ds(s * RPS, RPS)])

    return k(dst_rs, onerows, zeros128)


def _sc_gcn(hs, src_rs, dst_rs, zeros128):
    @functools.partial(
        pl.kernel,
        out_type=jax.ShapeDtypeStruct((NC, NP, H), jnp.float32),
        mesh=_mesh(),
        scratch_types=[
            pltpu.VMEM((NCS, C), jnp.int32),
            pltpu.VMEM((NCS, C), jnp.int32),
            pltpu.VMEM((C, H), jnp.float32),
            pltpu.VMEM((C, H), jnp.float32),
            pltpu.VMEM_SHARED((NP, H), jnp.float32),
            pltpu.SemaphoreType.DMA,
            pltpu.SemaphoreType.DMA,
            pltpu.SemaphoreType.DMA,
            pltpu.SemaphoreType.DMA,
        ],
    )
    def k(hs_hbm, src_hbm, dst_hbm, z_hbm, out_hbm,
          sidx, didx, rowsA, rowsB, acc, semA, semB, semSA, semSB):
        c = lax.axis_index("c")
        s = lax.axis_index("s")
        w = _wid()
        pltpu.sync_copy(z_hbm, acc.at[pl.ds(s * RPS, RPS)])
        plsc.subcore_barrier()

        def slab(m, carry):
            pltpu.sync_copy(src_hbm.at[w, m], sidx)
            pltpu.sync_copy(dst_hbm.at[w, m], didx)

            def body(kk, cy):
                a = 2 * kk
                b = 2 * kk + 1
                gA = pltpu.async_copy(hs_hbm.at[sidx.at[a]], rowsA, semA)
                gB = pltpu.async_copy(hs_hbm.at[sidx.at[b]], rowsB, semB)
                gA.wait()
                sA = pltpu.async_copy(rowsA, acc.at[didx.at[a]], semSA,
                                      add=True)
                gB.wait()
                sB = pltpu.async_copy(rowsB, acc.at[didx.at[b]], semSB,
                                      add=True)
                sA.wait()
                sB.wait()
                return cy

            lax.fori_loop(0, NCS // 2, body, 0)
            pltpu.async_copy(hs_hbm.at[sidx.at[NCS - 1]], rowsA, semA).wait()
            pltpu.sync_copy(rowsA, acc.at[didx.at[NCS - 1]], add=True)
            return carry

        lax.fori_loop(0, NSL, slab, 0)
        plsc.subcore_barrier()
        pltpu.sync_copy(acc.at[pl.ds(s * RPS, RPS)],
                        out_hbm.at[c, pl.ds(s * RPS, RPS)])

    return k(hs, src_rs, dst_rs, zeros128)


def _sc_logits(q, kk, src_rs, dst_rs):
    scale = 1.0 / math.sqrt(float(H))

    @functools.partial(
        pl.kernel,
        out_type=(jax.ShapeDtypeStruct((E,), jnp.float32),
                  jax.ShapeDtypeStruct((NW * 16,), jnp.float32)),
        mesh=_mesh(),
        scratch_types=[
            pltpu.VMEM((NSL, NCS, C), jnp.int32),
            pltpu.VMEM((NSL, NCS, C), jnp.int32),
            pltpu.VMEM((C, H), jnp.float32),
            pltpu.VMEM((C, H), jnp.float32),
            pltpu.VMEM((C, H), jnp.float32),
            pltpu.VMEM((C, H), jnp.float32),
            pltpu.VMEM((C,), jnp.float32),
            pltpu.VMEM((16,), jnp.float32),
            pltpu.SemaphoreType.DMA,
            pltpu.SemaphoreType.DMA,
        ],
    )
    def k(q_hbm, k_hbm, src_hbm, dst_hbm, lg_out, mx_out,
          sidx, didx, qA, kA, qB, kB, lbuf, mbuf, semA, semB):
        w = _wid()
        pltpu.sync_copy(src_hbm.at[w], sidx)
        pltpu.sync_copy(dst_hbm.at[w], didx)
        iota = lax.iota(jnp.int32, 16)

        def compute(qrows, krows, j, mvec):
            def group(g, mv):
                lv0 = jnp.zeros((16,), jnp.float32)
                lv1 = jnp.zeros((16,), jnp.float32)
                for l in range(16):
                    i = g * 16 + l
                    acc0 = qrows[i, pl.ds(0, 16)] * krows[i, pl.ds(0, 16)]
                    acc1 = qrows[i, pl.ds(16, 16)] * krows[i, pl.ds(16, 16)]
                    for b in range(2, 8, 2):
                        acc0 = acc0 + (qrows[i, pl.ds(b * 16, 16)] *
                                       krows[i, pl.ds(b * 16, 16)])
                        acc1 = acc1 + (qrows[i, pl.ds((b + 1) * 16, 16)] *
                                       krows[i, pl.ds((b + 1) * 16, 16)])
                    acc = acc0 + acc1
                    acc = acc + jnp.flip(acc)
                    sc0 = ((acc[0] + acc[1]) + (acc[2] + acc[3])) + \
                          ((acc[4] + acc[5]) + (acc[6] + acc[7]))
                    if l % 2 == 0:
                        lv0 = jnp.where(iota == l, sc0, lv0)
                    else:
                        lv1 = jnp.where(iota == l, sc0, lv1)
                lv = (lv0 + lv1) * scale
                lbuf[pl.ds(g * 16, 16)] = lv
                return jnp.maximum(mv, lv)

            mvec = lax.fori_loop(0, G16, group, mvec)
            pltpu.sync_copy(lbuf, lg_out.at[pl.ds(w * EPW + j * C, C)])
            return mvec

        def chunk_pair(t, mvec):
            a = 2 * t
            b = 2 * t + 1
            gA1 = pltpu.async_copy(
                q_hbm.at[didx.at[a // NCS, a % NCS]], qA, semA)
            gA2 = pltpu.async_copy(
                k_hbm.at[sidx.at[a // NCS, a % NCS]], kA, semA)
            gB1 = pltpu.async_copy(
                q_hbm.at[didx.at[b // NCS, b % NCS]], qB, semB)
            gB2 = pltpu.async_copy(
                k_hbm.at[sidx.at[b // NCS, b % NCS]], kB, semB)
            gA1.wait()
            gA2.wait()
            mvec = compute(qA, kA, a, mvec)
            gB1.wait()
            gB2.wait()
            mvec = compute(qB, kB, b, mvec)
            return mvec

        mvec = lax.fori_loop(
            0, NCH // 2, chunk_pair, jnp.full((16,), -1e30, jnp.float32))
        j = NCH - 1
        pltpu.async_copy(q_hbm.at[didx.at[NSL - 1, NCS - 1]], qA, semA).wait()
        pltpu.async_copy(k_hbm.at[sidx.at[NSL - 1, NCS - 1]], kA, semA).wait()
        mvec = compute(qA, kA, j, mvec)
        mbuf[...] = mvec
        pltpu.sync_copy(mbuf, mx_out.at[pl.ds(w * 16, 16)])

    return k(q, kk, src_rs, dst_rs)


def _sc_agg(v, logits, maxes, src_rs, dst_rs, zeros128):
    @functools.partial(
        pl.kernel,
        out_type=(jax.ShapeDtypeStruct((NC, NP, H), jnp.float32),
                  jax.ShapeDtypeStruct((NC, NP, H), jnp.float32)),
        mesh=_mesh(),
        scratch_types=[
            pltpu.VMEM((NCS, C), jnp.int32),
            pltpu.VMEM((NCS, C), jnp.int32),
            pltpu.VMEM((C,), jnp.float32),
            pltpu.VMEM((C,), jnp.float32),
            pltpu.VMEM((C, H), jnp.float32),
            pltpu.VMEM((C, H), jnp.float32),
            pltpu.VMEM((NW * 16,), jnp.float32),
            pltpu.VMEM_SHARED((NP, H), jnp.float32),
            pltpu.SemaphoreType.DMA,
            pltpu.SemaphoreType.DMA,
            pltpu.SemaphoreType.DMA,
            pltpu.SemaphoreType.DMA,
            pltpu.SemaphoreType.DMA,
            pltpu.SemaphoreType.DMA,
        ],
    )
    def k(v_hbm, lg_hbm, mx_hbm, src_hbm, dst_hbm, z_hbm,
          num_out, den_out,
          sidx, didx, lgcA, lgcB, scbufA, scbufB, mx_v, acc,
          semA, semB, semLA, semLB, semSA, semSB):
        c = lax.axis_index("c")
        s = lax.axis_index("s")
        w = _wid()
        pltpu.sync_copy(z_hbm, acc.at[pl.ds(s * RPS, RPS)])
        pltpu.sync_copy(mx_hbm, mx_v)
        m = mx_v[pl.ds(0, 16)]
        for i in range(1, NW):
            m = jnp.maximum(m, mx_v[pl.ds(i * 16, 16)])
        g0 = jnp.maximum(jnp.maximum(m[0], m[1]), jnp.maximum(m[2], m[3]))
        g1 = jnp.maximum(jnp.maximum(m[4], m[5]), jnp.maximum(m[6], m[7]))
        g2 = jnp.maximum(jnp.maximum(m[8], m[9]), jnp.maximum(m[10], m[11]))
        g3 = jnp.maximum(jnp.maximum(m[12], m[13]), jnp.maximum(m[14], m[15]))
        gmax = jnp.maximum(jnp.maximum(g0, g1), jnp.maximum(g2, g3))
        iota = lax.iota(jnp.int32, 16)
        plsc.subcore_barrier()

        def scale(scbuf, lgc):
            def group(g, cy):
                lv = lgc[pl.ds(g * 16, 16)]
                exl = jnp.exp(lv - gmax)
                for l in range(16):
                    i = g * 16 + l
                    ce = exl[l]
                    row_v = scbuf.at[i]
                    for b in range(8):
                        row_v[pl.ds(b * 16, 16)] = (
                            row_v[pl.ds(b * 16, 16)] * ce)
                return cy

            lax.fori_loop(0, G16, group, 0)

        def slab(mm, carry):
            pltpu.sync_copy(src_hbm.at[w, mm], sidx)
            pltpu.sync_copy(dst_hbm.at[w, mm], didx)
            base = w * EPW + mm * NCS * C

            def pair(kk, cy):
                a = 2 * kk
                b = 2 * kk + 1
                gA = pltpu.async_copy(v_hbm.at[sidx.at[a]], scbufA, semA)
                lA = pltpu.async_copy(
                    lg_hbm.at[pl.ds(base + a * C, C)], lgcA, semLA)
                gB = pltpu.async_copy(v_hbm.at[sidx.at[b]], scbufB, semB)
                lB = pltpu.async_copy(
                    lg_hbm.at[pl.ds(base + b * C, C)], lgcB, semLB)
                gA.wait()
                lA.wait()
                scale(scbufA, lgcA)
                sA = pltpu.async_copy(scbufA, acc.at[didx.at[a]], semSA,
                                      add=True)
                gB.wait()
                lB.wait()
                scale(scbufB, lgcB)
                sB = pltpu.async_copy(scbufB, acc.at[didx.at[b]], semSB,
                                      add=True)
                sA.wait()
                sB.wait()
                return cy

            lax.fori_loop(0, NCS // 2, pair, 0)
            a = NCS - 1
            pltpu.async_copy(v_hbm.at[sidx.at[a]], scbufA, semA).wait()
            pltpu.sync_copy(lg_hbm.at[pl.ds(base + a * C, C)], lgcA)
            scale(scbufA, lgcA)
            pltpu.sync_copy(scbufA, acc.at[didx.at[a]], add=True)
            return carry

        lax.fori_loop(0, NSL, slab, 0)
        plsc.subcore_barrier()
        pltpu.sync_copy(acc.at[pl.ds(s * RPS, RPS)],
                        num_out.at[c, pl.ds(s * RPS, RPS)])
        plsc.subcore_barrier()
        pltpu.sync_copy(z_hbm, acc.at[pl.ds(s * RPS, RPS)])
        plsc.subcore_barrier()

        zv = jnp.zeros((16,), jnp.float32)

        def zrow(r, carry):
            rvA = scbufA.at[r]
            rvB = scbufB.at[r]
            for b in range(8):
                rvA[pl.ds(b * 16, 16)] = zv
                rvB[pl.ds(b * 16, 16)] = zv
            return carry

        lax.fori_loop(0, C, zrow, 0)

        def onehot(scbuf, lgc):
            def group(g, cy):
                lv = lgc[pl.ds(g * 16, 16)]
                exl = jnp.exp(lv - gmax)
                for l in range(16):
                    i = g * 16 + l
                    scbuf.at[i][pl.ds(0, 16)] = jnp.where(
                        iota == 0, exl[l], 0.0)
                return cy

            lax.fori_loop(0, G16, group, 0)

        def slab2(mm, carry):
            pltpu.sync_copy(dst_hbm.at[w, mm], didx)
            base = w * EPW + mm * NCS * C

            def pair(kk, cy):
                a = 2 * kk
                b = 2 * kk + 1
                lA = pltpu.async_copy(
                    lg_hbm.at[pl.ds(base + a * C, C)], lgcA, semLA)
                lB = pltpu.async_copy(
                    lg_hbm.at[pl.ds(base + b * C, C)], lgcB, semLB)
                lA.wait()
                onehot(scbufA, lgcA)
                sA = pltpu.async_copy(scbufA, acc.at[didx.at[a]], semSA,
                                      add=True)
                lB.wait()
                onehot(scbufB, lgcB)
                sB = pltpu.async_copy(scbufB, acc.at[didx.at[b]], semSB,
                                      add=True)
                sA.wait()
                sB.wait()
                return cy

            lax.fori_loop(0, NCS // 2, pair, 0)
            a = NCS - 1
            pltpu.sync_copy(lg_hbm.at[pl.ds(base + a * C, C)], lgcA)
            onehot(scbufA, lgcA)
            pltpu.sync_copy(scbufA, acc.at[didx.at[a]], add=True)
            return carry

        lax.fori_loop(0, NSL, slab2, 0)
        plsc.subcore_barrier()
        pltpu.sync_copy(acc.at[pl.ds(s * RPS, RPS)],
                        den_out.at[c, pl.ds(s * RPS, RPS)])

    return k(v, logits, maxes, src_rs, dst_rs, zeros128)


def _row_spec(cols):
    return pl.BlockSpec((RB, cols), lambda i: (i, 0))


def _full_spec(shape):
    nd = len(shape)
    if nd == 2:
        return pl.BlockSpec(shape, lambda i: (0, 0))
    return pl.BlockSpec(shape, lambda i: (0,) * nd)


def _stack_spec(cols):
    return pl.BlockSpec((NC, RB, cols), lambda i: (0, i, 0))


def _tc1(x, Wgc1, deg2):
    def body(x_ref, w_ref, deg_ref, dinv_ref, hs_ref):
        d = deg_ref[0][:, :1] + deg_ref[1][:, :1] + 1.0
        dinv = lax.rsqrt(d)
        h = jnp.dot(x_ref[...], w_ref[...],
                    preferred_element_type=jnp.float32)
        dinv_ref[...] = dinv
        hs_ref[...] = h * dinv

    return pl.pallas_call(
        body,
        grid=(NP // RB,),
        in_specs=[_row_spec(D), _full_spec((D, H)), _stack_spec(H)],
        out_specs=[_row_spec(1), _row_spec(H)],
        out_shape=[jax.ShapeDtypeStruct((NP, 1), jnp.float32),
                   jax.ShapeDtypeStruct((NP, H), jnp.float32)],
    )(x, Wgc1, deg2)


def _tc2(agg1, hs1, dinv, Wgc2, bgc1):
    def body(agg_ref, hs_ref, dinv_ref, w_ref, b_ref, out_ref):
        dinv = dinv_ref[...]
        x2 = jax.nn.relu(dinv * (agg_ref[0] + agg_ref[1] + hs_ref[...])
                         + b_ref[...])
        h2 = jnp.dot(x2, w_ref[...], preferred_element_type=jnp.float32)
        out_ref[...] = h2 * dinv

    return pl.pallas_call(
        body,
        grid=(NP // RB,),
        in_specs=[_stack_spec(H), _row_spec(H), _row_spec(1),
                  _full_spec((H, H)), _full_spec((1, H))],
        out_specs=_row_spec(H),
        out_shape=jax.ShapeDtypeStruct((NP, H), jnp.float32),
    )(agg1, hs1, dinv, Wgc2, bgc1)


def _tc3(agg2, hs2, dinv, bgc2, x, protos, Wq_a, bq_a, Wk_a, bk_a, Wv_a,
         bv_a, Wo_a, bo_a, Wf, bf, Wq1, bq1, Wk1, bk1, Wv1, bv1, Ws1, bs1):
    scale = 1.0 / math.sqrt(float(H))

    def body(agg_ref, hs_ref, dinv_ref, bgc2_ref, x_ref, pr_ref,
             wqa_ref, bqa_ref, wka_ref, bka_ref, wva_ref, bva_ref,
             woa_ref, boa_ref, wf_ref, bf_ref,
             wq1_ref, bq1_ref, wk1_ref, bk1_ref, wv1_ref, bv1_ref,
             ws1_ref, bs1_ref,
             q1_ref, k1_ref, v1_ref, s1_ref):
        dinv = dinv_ref[...]
        x3 = jax.nn.relu(dinv * (agg_ref[0] + agg_ref[1] + hs_ref[...])
                         + bgc2_ref[...])
        qa = jnp.dot(x3, wqa_ref[...],
                     preferred_element_type=jnp.float32) + bqa_ref[...]
        pr = pr_ref[...]
        ka = jnp.dot(pr, wka_ref[...],
                     preferred_element_type=jnp.float32) + bka_ref[...]
        va = jnp.dot(pr, wva_ref[...],
                     preferred_element_type=jnp.float32) + bva_ref[...]
        lg = lax.dot_general(qa, ka, (((1,), (1,)), ((), ())),
                             preferred_element_type=jnp.float32) * scale
        lg = lg - jnp.max(lg, axis=-1, keepdims=True)
        ex = jnp.exp(lg)
        attn = ex / jnp.sum(ex, axis=-1, keepdims=True)
        nr = jnp.dot(jnp.dot(attn, va, preferred_element_type=jnp.float32),
                     woa_ref[...],
                     preferred_element_type=jnp.float32) + boa_ref[...]
        ftb = jnp.dot(x_ref[...], wf_ref[...],
                      preferred_element_type=jnp.float32) + bf_ref[...]
        z = jnp.concatenate([ftb, nr], axis=-1)
        q1_ref[...] = jnp.dot(z, wq1_ref[...],
                              preferred_element_type=jnp.float32) + bq1_ref[...]
        k1_ref[...] = jnp.dot(z, wk1_ref[...],
                              preferred_element_type=jnp.float32) + bk1_ref[...]
        v1_ref[...] = jnp.dot(z, wv1_ref[...],
                              preferred_element_type=jnp.float32) + bv1_ref[...]
        s1_ref[...] = jnp.dot(z, ws1_ref[...],
                              preferred_element_type=jnp.float32) + bs1_ref[...]

    z_in = FT + H
    return pl.pallas_call(
        body,
        grid=(NP // RB,),
        in_specs=[_stack_spec(H), _row_spec(H), _row_spec(1),
                  _full_spec((1, H)), _row_spec(D), _full_spec((P, H)),
                  _full_spec((H, H)), _full_spec((1, H)),
                  _full_spec((H, H)), _full_spec((1, H)),
                  _full_spec((H, H)), _full_spec((1, H)),
                  _full_spec((H, H)), _full_spec((1, H)),
                  _full_spec((D, FT)), _full_spec((1, FT)),
                  _full_spec((z_in, H)), _full_spec((1, H)),
                  _full_spec((z_in, H)), _full_spec((1, H)),
                  _full_spec((z_in, H)), _full_spec((1, H)),
                  _full_spec((z_in, H)), _full_spec((1, H))],
        out_specs=[_row_spec(H)] * 4,
        out_shape=[jax.ShapeDtypeStruct((NP, H), jnp.float32)] * 4,
    )(agg2, hs2, dinv, bgc2, x, protos, Wq_a, bq_a, Wk_a, bk_a, Wv_a, bv_a,
      Wo_a, bo_a, Wf, bf, Wq1, bq1, Wk1, bk1, Wv1, bv1, Ws1, bs1)


def _ln_block(z, g, b):
    mu = jnp.mean(z, axis=-1, keepdims=True)
    var = jnp.mean((z - mu) ** 2, axis=-1, keepdims=True)
    return g * (z - mu) / jnp.sqrt(var + 1e-5) + b


def _tc4(num1, den1, skip1, g1, be1, Wq2, bq2, Wk2, bk2, Wv2, bv2, Ws2, bs2):
    def body(num_ref, den_ref, sk_ref, g_ref, b_ref,
             wq_ref, bq_ref, wk_ref, bk_ref, wv_ref, bv_ref,
             ws_ref, bs_ref, q_ref, k_ref, v_ref, s_ref):
        num = num_ref[0] + num_ref[1]
        den = den_ref[0][:, :1] + den_ref[1][:, :1]
        z = num / (den + 1e-16) + sk_ref[...]
        z = _ln_block(z, g_ref[...], b_ref[...])
        q_ref[...] = jnp.dot(z, wq_ref[...],
                             preferred_element_type=jnp.float32) + bq_ref[...]
        k_ref[...] = jnp.dot(z, wk_ref[...],
                             preferred_element_type=jnp.float32) + bk_ref[...]
        v_ref[...] = jnp.dot(z, wv_ref[...],
                             preferred_element_type=jnp.float32) + bv_ref[...]
        s_ref[...] = jnp.dot(z, ws_ref[...],
                             preferred_element_type=jnp.float32) + bs_ref[...]

    return pl.pallas_call(
        body,
        grid=(NP // RB,),
        in_specs=[_stack_spec(H), _stack_spec(H), _row_spec(H),
                  _full_spec((1, H)), _full_spec((1, H)),
                  _full_spec((H, H)), _full_spec((1, H)),
                  _full_spec((H, H)), _full_spec((1, H)),
                  _full_spec((H, H)), _full_spec((1, H)),
                  _full_spec((H, H)), _full_spec((1, H))],
        out_specs=[_row_spec(H)] * 4,
        out_shape=[jax.ShapeDtypeStruct((NP, H), jnp.float32)] * 4,
    )(num1, den1, skip1, g1, be1, Wq2, bq2, Wk2, bk2, Wv2, bv2, Ws2, bs2)


def _tc5(num2, den2, skip2, g2, be2, Wm0, bm0, Wm1, bm1):
    def body(num_ref, den_ref, sk_ref, g_ref, b_ref, wm0_ref, bm0_ref,
             wm1_ref, bm1_ref, out_ref):
        num = num_ref[0] + num_ref[1]
        den = den_ref[0][:, :1] + den_ref[1][:, :1]
        z = num / (den + 1e-16) + sk_ref[...]
        z = _ln_block(z, g_ref[...], b_ref[...])
        y = jnp.dot(z, wm0_ref[...],
                    preferred_element_type=jnp.float32) + bm0_ref[...]
        y = 0.5 * y * (1.0 + lax.erf(y / math.sqrt(2.0)))
        y = jnp.dot(y, wm1_ref[...],
                    preferred_element_type=jnp.float32) + bm1_ref[...]
        out_ref[...] = jax.nn.sigmoid(y)

    return pl.pallas_call(
        body,
        grid=(NP // RB,),
        in_specs=[_stack_spec(H), _stack_spec(H), _row_spec(H),
                  _full_spec((1, H)), _full_spec((1, H)),
                  _full_spec((H, H // 2)), _full_spec((1, H // 2)),
                  _full_spec((H // 2, 1)), _full_spec((1, 1))],
        out_specs=_row_spec(1),
        out_shape=jax.ShapeDtypeStruct((NP, 1), jnp.float32),
    )(num2, den2, skip2, g2, be2, Wm0, bm0, Wm1, bm1)


def kernel(x, edge_index, prototypes, Wgc1, bgc1, Wgc2, bgc2, Wq_a, bq_a,
           Wk_a, bk_a, Wv_a, bv_a, Wo_a, bo_a, Wf, bf, Wq1, bq1, Wk1, bk1,
           Wv1, bv1, Ws1, bs1, g1, be1, Wq2, bq2, Wk2, bk2, Wv2, bv2, Ws2,
           bs2, g2, be2, Wm0, bm0, Wm1, bm1):
    src_rs = edge_index[0].reshape(NW, NSL, NCS, C)
    dst_rs = edge_index[1].reshape(NW, NSL, NCS, C)
    x = jnp.pad(x, ((0, NP - N), (0, 0)))
    onerows = jnp.tile(
        (lax.iota(jnp.int32, H) == 0).astype(jnp.float32)[None, :], (C, 1))
    zeros128 = jnp.zeros((RPS, H), jnp.float32)

    r2 = lambda b: b.reshape(1, -1)

    deg2 = _sc_deg(dst_rs, onerows, zeros128)
    dinv, hs1 = _tc1(x, Wgc1, deg2)
    agg1 = _sc_gcn(hs1, src_rs, dst_rs, zeros128)
    hs2 = _tc2(agg1, hs1, dinv, Wgc2, r2(bgc1))
    agg2 = _sc_gcn(hs2, src_rs, dst_rs, zeros128)
    q1, k1, v1, s1 = _tc3(
        agg2, hs2, dinv, r2(bgc2), x, prototypes,
        Wq_a, r2(bq_a), Wk_a, r2(bk_a), Wv_a, r2(bv_a), Wo_a, r2(bo_a),
        Wf, r2(bf), Wq1, r2(bq1), Wk1, r2(bk1), Wv1, r2(bv1), Ws1, r2(bs1))
    lg1, mx1 = _sc_logits(q1, k1, src_rs, dst_rs)
    num1, den1 = _sc_agg(v1, lg1, mx1, src_rs, dst_rs, zeros128)
    q2, k2, v2, s2 = _tc4(
        num1, den1, s1, r2(g1), r2(be1),
        Wq2, r2(bq2), Wk2, r2(bk2), Wv2, r2(bv2), Ws2, r2(bs2))
    lg2, mx2 = _sc_logits(q2, k2, src_rs, dst_rs)
    num2, den2 = _sc_agg(v2, lg2, mx2, src_rs, dst_rs, zeros128)
    out = _tc5(num2, den2, s2, r2(g2), r2(be2), Wm0, r2(bm0), Wm1, r2(bm1))
    return out[:N]


# speedup vs baseline: 1.0097x; 1.0097x over previous
import functools
import math

import jax
import jax.numpy as jnp
from jax import lax
from jax.experimental import pallas as pl
from jax.experimental.pallas import tpu as pltpu
from jax.experimental.pallas import tpu_sc as plsc

N = 10000
E = 320000
D = 128
H = 128
FT = 64
P = 20

NP = 10240

NC = 2
NS = 16
NW = NC * NS
C = 80
NCH = E // NW // C
EPW = NCH * C
RPS = NP // NS
G16 = C // 16
NSL = 5
NCS = NCH // NSL

RB = 320


def _mesh():
    return plsc.VectorSubcoreMesh(core_axis_name="c", subcore_axis_name="s")


def _wid():
    return lax.axis_index("s") * NC + lax.axis_index("c")


def _sc_deg(dst_rs, onerows, zeros128):
    @functools.partial(
        pl.kernel,
        out_type=jax.ShapeDtypeStruct((NC, NP, H), jnp.float32),
        mesh=_mesh(),
        scratch_types=[
            pltpu.VMEM((NSL, NCS, C), jnp.int32),
            pltpu.VMEM((C, H), jnp.float32),
            pltpu.VMEM_SHARED((NP, H), jnp.float32),
            pltpu.SemaphoreType.DMA,
            pltpu.SemaphoreType.DMA,
        ],
    )
    def k(dst_hbm, ones_hbm, z_hbm, out_hbm, didx, ones_v, acc, semA, semB):
        c = lax.axis_index("c")
        s = lax.axis_index("s")
        w = _wid()
        pltpu.sync_copy(z_hbm, acc.at[pl.ds(s * RPS, RPS)])
        pltpu.sync_copy(ones_hbm, ones_v)
        pltpu.sync_copy(dst_hbm.at[w], didx)
        plsc.subcore_barrier()

        def body(t, carry):
            a = 2 * t
            b = 2 * t + 1
            sA = pltpu.async_copy(
                ones_v, acc.at[didx.at[a // NCS, a % NCS]], semA, add=True)
            sB = pltpu.async_copy(
                ones_v, acc.at[didx.at[b // NCS, b % NCS]], semB, add=True)
            sA.wait()
            sB.wait()
            return carry

        lax.fori_loop(0, NCH // 2, body, 0)
        pltpu.sync_copy(
            ones_v, acc.at[didx.at[NSL - 1, NCS - 1]], add=True)
        plsc.subcore_barrier()
        pltpu.sync_copy(acc.at[pl.ds(s * RPS, RPS)],
                        out_hbm.at[c, pl.ds(s * RPS, RPS)])

    return k(dst_rs, onerows, zeros128)


def _sc_gcn(hs, src_rs, dst_rs, zeros128):
    @functools.partial(
        pl.kernel,
        out_type=jax.ShapeDtypeStruct((NC, NP, H), jnp.float32),
        mesh=_mesh(),
        scratch_types=[
            pltpu.VMEM((NCS, C), jnp.int32),
            pltpu.VMEM((NCS, C), jnp.int32),
            pltpu.VMEM((C, H), jnp.float32),
            pltpu.VMEM((C, H), jnp.float32),
            pltpu.VMEM_SHARED((NP, H), jnp.float32),
            pltpu.SemaphoreType.DMA,
            pltpu.SemaphoreType.DMA,
            pltpu.SemaphoreType.DMA,
            pltpu.SemaphoreType.DMA,
        ],
    )
    def k(hs_hbm, src_hbm, dst_hbm, z_hbm, out_hbm,
          sidx, didx, rowsA, rowsB, acc, semA, semB, semSA, semSB):
        c = lax.axis_index("c")
        s = lax.axis_index("s")
        w = _wid()
        pltpu.sync_copy(z_hbm, acc.at[pl.ds(s * RPS, RPS)])
        plsc.subcore_barrier()

        def slab(m, carry):
            pltpu.sync_copy(src_hbm.at[w, m], sidx)
            pltpu.sync_copy(dst_hbm.at[w, m], didx)

            def body(kk, cy):
                a = 2 * kk
                b = 2 * kk + 1
                gA = pltpu.async_copy(hs_hbm.at[sidx.at[a]], rowsA, semA)
                gB = pltpu.async_copy(hs_hbm.at[sidx.at[b]], rowsB, semB)
                gA.wait()
                sA = pltpu.async_copy(rowsA, acc.at[didx.at[a]], semSA,
                                      add=True)
                gB.wait()
                sB = pltpu.async_copy(rowsB, acc.at[didx.at[b]], semSB,
                                      add=True)
                sA.wait()
                sB.wait()
                return cy

            lax.fori_loop(0, NCS // 2, body, 0)
            pltpu.async_copy(hs_hbm.at[sidx.at[NCS - 1]], rowsA, semA).wait()
            pltpu.sync_copy(rowsA, acc.at[didx.at[NCS - 1]], add=True)
            return carry

        lax.fori_loop(0, NSL, slab, 0)
        plsc.subcore_barrier()
        pltpu.sync_copy(acc.at[pl.ds(s * RPS, RPS)],
                        out_hbm.at[c, pl.ds(s * RPS, RPS)])

    return k(hs, src_rs, dst_rs, zeros128)


def _sc_logits(q, kk, src_rs, dst_rs):
    scale = 1.0 / math.sqrt(float(H))

    @functools.partial(
        pl.kernel,
        out_type=(jax.ShapeDtypeStruct((E,), jnp.float32),
                  jax.ShapeDtypeStruct((NW * 16,), jnp.float32)),
        mesh=_mesh(),
        scratch_types=[
            pltpu.VMEM((NSL, NCS, C), jnp.int32),
            pltpu.VMEM((NSL, NCS, C), jnp.int32),
            pltpu.VMEM((C, H), jnp.float32),
            pltpu.VMEM((C, H), jnp.float32),
            pltpu.VMEM((C, H), jnp.float32),
            pltpu.VMEM((C, H), jnp.float32),
            pltpu.VMEM((C,), jnp.float32),
            pltpu.VMEM((16,), jnp.float32),
            pltpu.SemaphoreType.DMA,
            pltpu.SemaphoreType.DMA,
        ],
    )
    def k(q_hbm, k_hbm, src_hbm, dst_hbm, lg_out, mx_out,
          sidx, didx, qA, kA, qB, kB, lbuf, mbuf, semA, semB):
        w = _wid()
        pltpu.sync_copy(src_hbm.at[w], sidx)
        pltpu.sync_copy(dst_hbm.at[w], didx)
        iota = lax.iota(jnp.int32, 16)

        def compute(qrows, krows, j, mvec):
            def group(g, mv):
                lv = jnp.zeros((16,), jnp.float32)
                for l in range(16):
                    i = g * 16 + l
                    acc = qrows[i, pl.ds(0, 16)] * krows[i, pl.ds(0, 16)]
                    for b in range(1, 8):
                        acc = acc + (qrows[i, pl.ds(b * 16, 16)] *
                                     krows[i, pl.ds(b * 16, 16)])
                    acc = acc + jnp.flip(acc)
                    sc0 = ((acc[0] + acc[1]) + (acc[2] + acc[3])) + \
                          ((acc[4] + acc[5]) + (acc[6] + acc[7]))
                    lv = jnp.where(iota == l, sc0, lv)
                lv = lv * scale
                lbuf[pl.ds(g * 16, 16)] = lv
                return jnp.maximum(mv, lv)

            mvec = lax.fori_loop(0, G16, group, mvec)
            pltpu.sync_copy(lbuf, lg_out.at[pl.ds(w * EPW + j * C, C)])
            return mvec

        def chunk_pair(t, mvec):
            a = 2 * t
            b = 2 * t + 1
            gA1 = pltpu.async_copy(
                q_hbm.at[didx.at[a // NCS, a % NCS]], qA, semA)
            gA2 = pltpu.async_copy(
                k_hbm.at[sidx.at[a // NCS, a % NCS]], kA, semA)
            gB1 = pltpu.async_copy(
                q_hbm.at[didx.at[b // NCS, b % NCS]], qB, semB)
            gB2 = pltpu.async_copy(
                k_hbm.at[sidx.at[b // NCS, b % NCS]], kB, semB)
            gA1.wait()
            gA2.wait()
            mvec = compute(qA, kA, a, mvec)
            gB1.wait()
            gB2.wait()
            mvec = compute(qB, kB, b, mvec)
            return mvec

        mvec = lax.fori_loop(
            0, NCH // 2, chunk_pair, jnp.full((16,), -1e30, jnp.float32))
        j = NCH - 1
        pltpu.async_copy(q_hbm.at[didx.at[NSL - 1, NCS - 1]], qA, semA).wait()
        pltpu.async_copy(k_hbm.at[sidx.at[NSL - 1, NCS - 1]], kA, semA).wait()
        mvec = compute(qA, kA, j, mvec)
        mbuf[...] = mvec
        pltpu.sync_copy(mbuf, mx_out.at[pl.ds(w * 16, 16)])

    return k(q, kk, src_rs, dst_rs)


def _sc_agg(v, logits, maxes, src_rs, dst_rs, zeros128):
    @functools.partial(
        pl.kernel,
        out_type=(jax.ShapeDtypeStruct((NC, NP, H), jnp.float32),
                  jax.ShapeDtypeStruct((NC, NP, H), jnp.float32)),
        mesh=_mesh(),
        scratch_types=[
            pltpu.VMEM((NCS, C), jnp.int32),
            pltpu.VMEM((NCS, C), jnp.int32),
            pltpu.VMEM((C,), jnp.float32),
            pltpu.VMEM((C,), jnp.float32),
            pltpu.VMEM((C, H), jnp.float32),
            pltpu.VMEM((C, H), jnp.float32),
            pltpu.VMEM((NW * 16,), jnp.float32),
            pltpu.VMEM_SHARED((NP, H), jnp.float32),
            pltpu.SemaphoreType.DMA,
            pltpu.SemaphoreType.DMA,
            pltpu.SemaphoreType.DMA,
            pltpu.SemaphoreType.DMA,
            pltpu.SemaphoreType.DMA,
            pltpu.SemaphoreType.DMA,
        ],
    )
    def k(v_hbm, lg_hbm, mx_hbm, src_hbm, dst_hbm, z_hbm,
          num_out, den_out,
          sidx, didx, lgcA, lgcB, scbufA, scbufB, mx_v, acc,
          semA, semB, semLA, semLB, semSA, semSB):
        c = lax.axis_index("c")
        s = lax.axis_index("s")
        w = _wid()
        pltpu.sync_copy(z_hbm, acc.at[pl.ds(s * RPS, RPS)])
        pltpu.sync_copy(mx_hbm, mx_v)
        m = mx_v[pl.ds(0, 16)]
        for i in range(1, NW):
            m = jnp.maximum(m, mx_v[pl.ds(i * 16, 16)])
        g0 = jnp.maximum(jnp.maximum(m[0], m[1]), jnp.maximum(m[2], m[3]))
        g1 = jnp.maximum(jnp.maximum(m[4], m[5]), jnp.maximum(m[6], m[7]))
        g2 = jnp.maximum(jnp.maximum(m[8], m[9]), jnp.maximum(m[10], m[11]))
        g3 = jnp.maximum(jnp.maximum(m[12], m[13]), jnp.maximum(m[14], m[15]))
        gmax = jnp.maximum(jnp.maximum(g0, g1), jnp.maximum(g2, g3))
        iota = lax.iota(jnp.int32, 16)
        plsc.subcore_barrier()

        def scale(scbuf, lgc):
            def group(g, cy):
                lv = lgc[pl.ds(g * 16, 16)]
                exl = jnp.exp(lv - gmax)
                for l in range(16):
                    i = g * 16 + l
                    ce = exl[l]
                    row_v = scbuf.at[i]
                    for b in range(8):
                        row_v[pl.ds(b * 16, 16)] = (
                            row_v[pl.ds(b * 16, 16)] * ce)
                return cy

            lax.fori_loop(0, G16, group, 0)

        def slab(mm, carry):
            pltpu.sync_copy(src_hbm.at[w, mm], sidx)
            pltpu.sync_copy(dst_hbm.at[w, mm], didx)
            base = w * EPW + mm * NCS * C

            def pair(kk, cy):
                a = 2 * kk
                b = 2 * kk + 1
                gA = pltpu.async_copy(v_hbm.at[sidx.at[a]], scbufA, semA)
                lA = pltpu.async_copy(
                    lg_hbm.at[pl.ds(base + a * C, C)], lgcA, semLA)
                gB = pltpu.async_copy(v_hbm.at[sidx.at[b]], scbufB, semB)
                lB = pltpu.async_copy(
                    lg_hbm.at[pl.ds(base + b * C, C)], lgcB, semLB)
                gA.wait()
                lA.wait()
                scale(scbufA, lgcA)
                sA = pltpu.async_copy(scbufA, acc.at[didx.at[a]], semSA,
                                      add=True)
                gB.wait()
                lB.wait()
                scale(scbufB, lgcB)
                sB = pltpu.async_copy(scbufB, acc.at[didx.at[b]], semSB,
                                      add=True)
                sA.wait()
                sB.wait()
                return cy

            lax.fori_loop(0, NCS // 2, pair, 0)
            a = NCS - 1
            pltpu.async_copy(v_hbm.at[sidx.at[a]], scbufA, semA).wait()
            pltpu.sync_copy(lg_hbm.at[pl.ds(base + a * C, C)], lgcA)
            scale(scbufA, lgcA)
            pltpu.sync_copy(scbufA, acc.at[didx.at[a]], add=True)
            return carry

        lax.fori_loop(0, NSL, slab, 0)
        plsc.subcore_barrier()
        pltpu.sync_copy(acc.at[pl.ds(s * RPS, RPS)],
                        num_out.at[c, pl.ds(s * RPS, RPS)])
        plsc.subcore_barrier()
        pltpu.sync_copy(z_hbm, acc.at[pl.ds(s * RPS, RPS)])
        plsc.subcore_barrier()

        zv = jnp.zeros((16,), jnp.float32)

        def zrow(r, carry):
            rvA = scbufA.at[r]
            rvB = scbufB.at[r]
            for b in range(8):
                rvA[pl.ds(b * 16, 16)] = zv
                rvB[pl.ds(b * 16, 16)] = zv
            return carry

        lax.fori_loop(0, C, zrow, 0)

        def onehot(scbuf, lgc):
            def group(g, cy):
                lv = lgc[pl.ds(g * 16, 16)]
                exl = jnp.exp(lv - gmax)
                for l in range(16):
                    i = g * 16 + l
                    scbuf.at[i][pl.ds(0, 16)] = jnp.where(
                        iota == 0, exl[l], 0.0)
                return cy

            lax.fori_loop(0, G16, group, 0)

        def slab2(mm, carry):
            pltpu.sync_copy(dst_hbm.at[w, mm], didx)
            base = w * EPW + mm * NCS * C

            def pair(kk, cy):
                a = 2 * kk
                b = 2 * kk + 1
                lA = pltpu.async_copy(
                    lg_hbm.at[pl.ds(base + a * C, C)], lgcA, semLA)
                lB = pltpu.async_copy(
                    lg_hbm.at[pl.ds(base + b * C, C)], lgcB, semLB)
                lA.wait()
                onehot(scbufA, lgcA)
                sA = pltpu.async_copy(scbufA, acc.at[didx.at[a]], semSA,
                                      add=True)
                lB.wait()
                onehot(scbufB, lgcB)
                sB = pltpu.async_copy(scbufB, acc.at[didx.at[b]], semSB,
                                      add=True)
                sA.wait()
                sB.wait()
                return cy

            lax.fori_loop(0, NCS // 2, pair, 0)
            a = NCS - 1
            pltpu.sync_copy(lg_hbm.at[pl.ds(base + a * C, C)], lgcA)
            onehot(scbufA, lgcA)
            pltpu.sync_copy(scbufA, acc.at[didx.at[a]], add=True)
            return carry

        lax.fori_loop(0, NSL, slab2, 0)
        plsc.subcore_barrier()
        pltpu.sync_copy(acc.at[pl.ds(s * RPS, RPS)],
                        den_out.at[c, pl.ds(s * RPS, RPS)])

    return k(v, logits, maxes, src_rs, dst_rs, zeros128)


def _row_spec(cols):
    return pl.BlockSpec((RB, cols), lambda i: (i, 0))


def _full_spec(shape):
    nd = len(shape)
    if nd == 2:
        return pl.BlockSpec(shape, lambda i: (0, 0))
    return pl.BlockSpec(shape, lambda i: (0,) * nd)


def _stack_spec(cols):
    return pl.BlockSpec((NC, RB, cols), lambda i: (0, i, 0))


def _tc1(x, Wgc1, deg2):
    def body(x_ref, w_ref, deg_ref, dinv_ref, hs_ref):
        d = deg_ref[0][:, :1] + deg_ref[1][:, :1] + 1.0
        dinv = lax.rsqrt(d)
        h = jnp.dot(x_ref[...], w_ref[...],
                    preferred_element_type=jnp.float32)
        dinv_ref[...] = dinv
        hs_ref[...] = h * dinv

    return pl.pallas_call(
        body,
        grid=(NP // RB,),
        in_specs=[_row_spec(D), _full_spec((D, H)), _stack_spec(H)],
        out_specs=[_row_spec(1), _row_spec(H)],
        out_shape=[jax.ShapeDtypeStruct((NP, 1), jnp.float32),
                   jax.ShapeDtypeStruct((NP, H), jnp.float32)],
    )(x, Wgc1, deg2)


def _tc2(agg1, hs1, dinv, Wgc2, bgc1):
    def body(agg_ref, hs_ref, dinv_ref, w_ref, b_ref, out_ref):
        dinv = dinv_ref[...]
        x2 = jax.nn.relu(dinv * (agg_ref[0] + agg_ref[1] + hs_ref[...])
                         + b_ref[...])
        h2 = jnp.dot(x2, w_ref[...], preferred_element_type=jnp.float32)
        out_ref[...] = h2 * dinv

    return pl.pallas_call(
        body,
        grid=(NP // RB,),
        in_specs=[_stack_spec(H), _row_spec(H), _row_spec(1),
                  _full_spec((H, H)), _full_spec((1, H))],
        out_specs=_row_spec(H),
        out_shape=jax.ShapeDtypeStruct((NP, H), jnp.float32),
    )(agg1, hs1, dinv, Wgc2, bgc1)


def _tc3(agg2, hs2, dinv, bgc2, x, protos, Wq_a, bq_a, Wk_a, bk_a, Wv_a,
         bv_a, Wo_a, bo_a, Wf, bf, Wq1, bq1, Wk1, bk1, Wv1, bv1, Ws1, bs1):
    scale = 1.0 / math.sqrt(float(H))

    def body(agg_ref, hs_ref, dinv_ref, bgc2_ref, x_ref, pr_ref,
             wqa_ref, bqa_ref, wka_ref, bka_ref, wva_ref, bva_ref,
             woa_ref, boa_ref, wf_ref, bf_ref,
             wq1_ref, bq1_ref, wk1_ref, bk1_ref, wv1_ref, bv1_ref,
             ws1_ref, bs1_ref,
             q1_ref, k1_ref, v1_ref, s1_ref):
        dinv = dinv_ref[...]
        x3 = jax.nn.relu(dinv * (agg_ref[0] + agg_ref[1] + hs_ref[...])
                         + bgc2_ref[...])
        qa = jnp.dot(x3, wqa_ref[...],
                     preferred_element_type=jnp.float32) + bqa_ref[...]
        pr = pr_ref[...]
        ka = jnp.dot(pr, wka_ref[...],
                     preferred_element_type=jnp.float32) + bka_ref[...]
        va = jnp.dot(pr, wva_ref[...],
                     preferred_element_type=jnp.float32) + bva_ref[...]
        lg = lax.dot_general(qa, ka, (((1,), (1,)), ((), ())),
                             preferred_element_type=jnp.float32) * scale
        lg = lg - jnp.max(lg, axis=-1, keepdims=True)
        ex = jnp.exp(lg)
        attn = ex / jnp.sum(ex, axis=-1, keepdims=True)
        nr = jnp.dot(jnp.dot(attn, va, preferred_element_type=jnp.float32),
                     woa_ref[...],
                     preferred_element_type=jnp.float32) + boa_ref[...]
        ftb = jnp.dot(x_ref[...], wf_ref[...],
                      preferred_element_type=jnp.float32) + bf_ref[...]
        z = jnp.concatenate([ftb, nr], axis=-1)
        q1_ref[...] = jnp.dot(z, wq1_ref[...],
                              preferred_element_type=jnp.float32) + bq1_ref[...]
        k1_ref[...] = jnp.dot(z, wk1_ref[...],
                              preferred_element_type=jnp.float32) + bk1_ref[...]
        v1_ref[...] = jnp.dot(z, wv1_ref[...],
                              preferred_element_type=jnp.float32) + bv1_ref[...]
        s1_ref[...] = jnp.dot(z, ws1_ref[...],
                              preferred_element_type=jnp.float32) + bs1_ref[...]

    z_in = FT + H
    return pl.pallas_call(
        body,
        grid=(NP // RB,),
        in_specs=[_stack_spec(H), _row_spec(H), _row_spec(1),
                  _full_spec((1, H)), _row_spec(D), _full_spec((P, H)),
                  _full_spec((H, H)), _full_spec((1, H)),
                  _full_spec((H, H)), _full_spec((1, H)),
                  _full_spec((H, H)), _full_spec((1, H)),
                  _full_spec((H, H)), _full_spec((1, H)),
                  _full_spec((D, FT)), _full_spec((1, FT)),
                  _full_spec((z_in, H)), _full_spec((1, H)),
                  _full_spec((z_in, H)), _full_spec((1, H)),
                  _full_spec((z_in, H)), _full_spec((1, H)),
                  _full_spec((z_in, H)), _full_spec((1, H))],
        out_specs=[_row_spec(H)] * 4,
        out_shape=[jax.ShapeDtypeStruct((NP, H), jnp.float32)] * 4,
    )(agg2, hs2, dinv, bgc2, x, protos, Wq_a, bq_a, Wk_a, bk_a, Wv_a, bv_a,
      Wo_a, bo_a, Wf, bf, Wq1, bq1, Wk1, bk1, Wv1, bv1, Ws1, bs1)


def _ln_block(z, g, b):
    mu = jnp.mean(z, axis=-1, keepdims=True)
    var = jnp.mean((z - mu) ** 2, axis=-1, keepdims=True)
    return g * (z - mu) / jnp.sqrt(var + 1e-5) + b


def _tc4(num1, den1, skip1, g1, be1, Wq2, bq2, Wk2, bk2, Wv2, bv2, Ws2, bs2):
    def body(num_ref, den_ref, sk_ref, g_ref, b_ref,
             wq_ref, bq_ref, wk_ref, bk_ref, wv_ref, bv_ref,
             ws_ref, bs_ref, q_ref, k_ref, v_ref, s_ref):
        num = num_ref[0] + num_ref[1]
        den = den_ref[0][:, :1] + den_ref[1][:, :1]
        z = num / (den + 1e-16) + sk_ref[...]
        z = _ln_block(z, g_ref[...], b_ref[...])
        q_ref[...] = jnp.dot(z, wq_ref[...],
                             preferred_element_type=jnp.float32) + bq_ref[...]
        k_ref[...] = jnp.dot(z, wk_ref[...],
                             preferred_element_type=jnp.float32) + bk_ref[...]
        v_ref[...] = jnp.dot(z, wv_ref[...],
                             preferred_element_type=jnp.float32) + bv_ref[...]
        s_ref[...] = jnp.dot(z, ws_ref[...],
                             preferred_element_type=jnp.float32) + bs_ref[...]

    return pl.pallas_call(
        body,
        grid=(NP // RB,),
        in_specs=[_stack_spec(H), _stack_spec(H), _row_spec(H),
                  _full_spec((1, H)), _full_spec((1, H)),
                  _full_spec((H, H)), _full_spec((1, H)),
                  _full_spec((H, H)), _full_spec((1, H)),
                  _full_spec((H, H)), _full_spec((1, H)),
                  _full_spec((H, H)), _full_spec((1, H))],
        out_specs=[_row_spec(H)] * 4,
        out_shape=[jax.ShapeDtypeStruct((NP, H), jnp.float32)] * 4,
    )(num1, den1, skip1, g1, be1, Wq2, bq2, Wk2, bk2, Wv2, bv2, Ws2, bs2)


def _tc5(num2, den2, skip2, g2, be2, Wm0, bm0, Wm1, bm1):
    def body(num_ref, den_ref, sk_ref, g_ref, b_ref, wm0_ref, bm0_ref,
             wm1_ref, bm1_ref, out_ref):
        num = num_ref[0] + num_ref[1]
        den = den_ref[0][:, :1] + den_ref[1][:, :1]
        z = num / (den + 1e-16) + sk_ref[...]
        z = _ln_block(z, g_ref[...], b_ref[...])
        y = jnp.dot(z, wm0_ref[...],
                    preferred_element_type=jnp.float32) + bm0_ref[...]
        y = 0.5 * y * (1.0 + lax.erf(y / math.sqrt(2.0)))
        y = jnp.dot(y, wm1_ref[...],
                    preferred_element_type=jnp.float32) + bm1_ref[...]
        out_ref[...] = jax.nn.sigmoid(y)

    return pl.pallas_call(
        body,
        grid=(NP // RB,),
        in_specs=[_stack_spec(H), _stack_spec(H), _row_spec(H),
                  _full_spec((1, H)), _full_spec((1, H)),
                  _full_spec((H, H // 2)), _full_spec((1, H // 2)),
                  _full_spec((H // 2, 1)), _full_spec((1, 1))],
        out_specs=_row_spec(1),
        out_shape=jax.ShapeDtypeStruct((NP, 1), jnp.float32),
    )(num2, den2, skip2, g2, be2, Wm0, bm0, Wm1, bm1)


def kernel(x, edge_index, prototypes, Wgc1, bgc1, Wgc2, bgc2, Wq_a, bq_a,
           Wk_a, bk_a, Wv_a, bv_a, Wo_a, bo_a, Wf, bf, Wq1, bq1, Wk1, bk1,
           Wv1, bv1, Ws1, bs1, g1, be1, Wq2, bq2, Wk2, bk2, Wv2, bv2, Ws2,
           bs2, g2, be2, Wm0, bm0, Wm1, bm1):
    src_rs = edge_index[0].reshape(NW, NSL, NCS, C)
    dst_rs = edge_index[1].reshape(NW, NSL, NCS, C)
    x = jnp.pad(x, ((0, NP - N), (0, 0)))
    onerows = jnp.tile(
        (lax.iota(jnp.int32, H) == 0).astype(jnp.float32)[None, :], (C, 1))
    zeros128 = jnp.zeros((RPS, H), jnp.float32)

    r2 = lambda b: b.reshape(1, -1)

    deg2 = _sc_deg(dst_rs, onerows, zeros128)
    dinv, hs1 = _tc1(x, Wgc1, deg2)
    agg1 = _sc_gcn(hs1, src_rs, dst_rs, zeros128)
    hs2 = _tc2(agg1, hs1, dinv, Wgc2, r2(bgc1))
    agg2 = _sc_gcn(hs2, src_rs, dst_rs, zeros128)
    q1, k1, v1, s1 = _tc3(
        agg2, hs2, dinv, r2(bgc2), x, prototypes,
        Wq_a, r2(bq_a), Wk_a, r2(bk_a), Wv_a, r2(bv_a), Wo_a, r2(bo_a),
        Wf, r2(bf), Wq1, r2(bq1), Wk1, r2(bk1), Wv1, r2(bv1), Ws1, r2(bs1))
    lg1, mx1 = _sc_logits(q1, k1, src_rs, dst_rs)
    num1, den1 = _sc_agg(v1, lg1, mx1, src_rs, dst_rs, zeros128)
    q2, k2, v2, s2 = _tc4(
        num1, den1, s1, r2(g1), r2(be1),
        Wq2, r2(bq2), Wk2, r2(bk2), Wv2, r2(bv2), Ws2, r2(bs2))
    lg2, mx2 = _sc_logits(q2, k2, src_rs, dst_rs)
    num2, den2 = _sc_agg(v2, lg2, mx2, src_rs, dst_rs, zeros128)
    out = _tc5(num2, den2, s2, r2(g2), r2(be2), Wm0, r2(bm0), Wm1, r2(bm1))
    return out[:N]


# speedup vs baseline: 1.0130x; 1.0033x over previous
import functools
import math

import jax
import jax.numpy as jnp
from jax import lax
from jax.experimental import pallas as pl
from jax.experimental.pallas import tpu as pltpu
from jax.experimental.pallas import tpu_sc as plsc

N = 10000
E = 320000
D = 128
H = 128
FT = 64
P = 20

NP = 10240

NC = 2
NS = 16
NW = NC * NS
C = 80
NCH = E // NW // C
EPW = NCH * C
RPS = NP // NS
G16 = C // 16
NSL = 5
NCS = NCH // NSL

RB = 320


def _mesh():
    return plsc.VectorSubcoreMesh(core_axis_name="c", subcore_axis_name="s")


def _wid():
    return lax.axis_index("s") * NC + lax.axis_index("c")


def _sc_deg(dst_rs, onerows, zeros128):
    @functools.partial(
        pl.kernel,
        out_type=jax.ShapeDtypeStruct((NC, NP, H), jnp.float32),
        mesh=_mesh(),
        scratch_types=[
            pltpu.VMEM((NSL, NCS, C), jnp.int32),
            pltpu.VMEM((C, H), jnp.float32),
            pltpu.VMEM_SHARED((NP, H), jnp.float32),
            pltpu.SemaphoreType.DMA,
            pltpu.SemaphoreType.DMA,
        ],
    )
    def k(dst_hbm, ones_hbm, z_hbm, out_hbm, didx, ones_v, acc, semA, semB):
        c = lax.axis_index("c")
        s = lax.axis_index("s")
        w = _wid()
        pltpu.sync_copy(z_hbm, acc.at[pl.ds(s * RPS, RPS)])
        pltpu.sync_copy(ones_hbm, ones_v)
        pltpu.sync_copy(dst_hbm.at[w], didx)
        plsc.subcore_barrier()

        def body(t, carry):
            a = 2 * t
            b = 2 * t + 1
            sA = pltpu.async_copy(
                ones_v, acc.at[didx.at[a // NCS, a % NCS]], semA, add=True)
            sB = pltpu.async_copy(
                ones_v, acc.at[didx.at[b // NCS, b % NCS]], semB, add=True)
            sA.wait()
            sB.wait()
            return carry

        lax.fori_loop(0, NCH // 2, body, 0)
        pltpu.sync_copy(
            ones_v, acc.at[didx.at[NSL - 1, NCS - 1]], add=True)
        plsc.subcore_barrier()
        pltpu.sync_copy(acc.at[pl.ds(s * RPS, RPS)],
                        out_hbm.at[c, pl.ds(s * RPS, RPS)])

    return k(dst_rs, onerows, zeros128)


def _sc_gcn(hs, src_rs, dst_rs, zeros128):
    @functools.partial(
        pl.kernel,
        out_type=jax.ShapeDtypeStruct((NC, NP, H), jnp.float32),
        mesh=_mesh(),
        scratch_types=[
            pltpu.VMEM((NCS, C), jnp.int32),
            pltpu.VMEM((NCS, C), jnp.int32),
            pltpu.VMEM((C, H), jnp.float32),
            pltpu.VMEM((C, H), jnp.float32),
            pltpu.VMEM_SHARED((NP, H), jnp.float32),
            pltpu.SemaphoreType.DMA,
            pltpu.SemaphoreType.DMA,
            pltpu.SemaphoreType.DMA,
            pltpu.SemaphoreType.DMA,
        ],
    )
    def k(hs_hbm, src_hbm, dst_hbm, z_hbm, out_hbm,
          sidx, didx, rowsA, rowsB, acc, semA, semB, semSA, semSB):
        c = lax.axis_index("c")
        s = lax.axis_index("s")
        w = _wid()
        pltpu.sync_copy(z_hbm, acc.at[pl.ds(s * RPS, RPS)])
        plsc.subcore_barrier()

        def slab(m, carry):
            pltpu.sync_copy(src_hbm.at[w, m], sidx)
            pltpu.sync_copy(dst_hbm.at[w, m], didx)

            def body(kk, cy):
                a = 2 * kk
                b = 2 * kk + 1
                gA = pltpu.async_copy(hs_hbm.at[sidx.at[a]], rowsA, semA)
                gB = pltpu.async_copy(hs_hbm.at[sidx.at[b]], rowsB, semB)
                gA.wait()
                sA = pltpu.async_copy(rowsA, acc.at[didx.at[a]], semSA,
                                      add=True)
                gB.wait()
                sB = pltpu.async_copy(rowsB, acc.at[didx.at[b]], semSB,
                                      add=True)
                sA.wait()
                sB.wait()
                return cy

            lax.fori_loop(0, NCS // 2, body, 0)
            pltpu.async_copy(hs_hbm.at[sidx.at[NCS - 1]], rowsA, semA).wait()
            pltpu.sync_copy(rowsA, acc.at[didx.at[NCS - 1]], add=True)
            return carry

        lax.fori_loop(0, NSL, slab, 0)
        plsc.subcore_barrier()
        pltpu.sync_copy(acc.at[pl.ds(s * RPS, RPS)],
                        out_hbm.at[c, pl.ds(s * RPS, RPS)])

    return k(hs, src_rs, dst_rs, zeros128)


def _sc_logits(q, kk, src_rs, dst_rs):
    scale = 1.0 / math.sqrt(float(H))

    @functools.partial(
        pl.kernel,
        out_type=(jax.ShapeDtypeStruct((E,), jnp.float32),
                  jax.ShapeDtypeStruct((NW * 16,), jnp.float32)),
        mesh=_mesh(),
        scratch_types=[
            pltpu.VMEM((NSL, NCS, C), jnp.int32),
            pltpu.VMEM((NSL, NCS, C), jnp.int32),
            pltpu.VMEM((C, H), jnp.float32),
            pltpu.VMEM((C, H), jnp.float32),
            pltpu.VMEM((C, H), jnp.float32),
            pltpu.VMEM((C, H), jnp.float32),
            pltpu.VMEM((C,), jnp.float32),
            pltpu.VMEM((C,), jnp.float32),
            pltpu.VMEM((16,), jnp.float32),
            pltpu.SemaphoreType.DMA,
            pltpu.SemaphoreType.DMA,
            pltpu.SemaphoreType.DMA,
            pltpu.SemaphoreType.DMA,
        ],
    )
    def k(q_hbm, k_hbm, src_hbm, dst_hbm, lg_out, mx_out,
          sidx, didx, qA, kA, qB, kB, lbufA, lbufB, mbuf,
          semA, semB, semOA, semOB):
        w = _wid()
        pltpu.sync_copy(src_hbm.at[w], sidx)
        pltpu.sync_copy(dst_hbm.at[w], didx)
        iota = lax.iota(jnp.int32, 16)

        def compute(qrows, krows, j, mvec, lbuf, semO):
            def group(g, mv):
                lv = jnp.zeros((16,), jnp.float32)
                for l in range(16):
                    i = g * 16 + l
                    acc = qrows[i, pl.ds(0, 16)] * krows[i, pl.ds(0, 16)]
                    for b in range(1, 8):
                        acc = acc + (qrows[i, pl.ds(b * 16, 16)] *
                                     krows[i, pl.ds(b * 16, 16)])
                    acc = acc + jnp.flip(acc)
                    sc0 = ((acc[0] + acc[1]) + (acc[2] + acc[3])) + \
                          ((acc[4] + acc[5]) + (acc[6] + acc[7]))
                    lv = jnp.where(iota == l, sc0, lv)
                lv = lv * scale
                lbuf[pl.ds(g * 16, 16)] = lv
                return jnp.maximum(mv, lv)

            mvec = lax.fori_loop(0, G16, group, mvec)
            st = pltpu.async_copy(
                lbuf, lg_out.at[pl.ds(w * EPW + j * C, C)], semO)
            return mvec, st

        def chunk_pair(t, mvec):
            a = 2 * t
            b = 2 * t + 1
            gA1 = pltpu.async_copy(
                q_hbm.at[didx.at[a // NCS, a % NCS]], qA, semA)
            gA2 = pltpu.async_copy(
                k_hbm.at[sidx.at[a // NCS, a % NCS]], kA, semA)
            gB1 = pltpu.async_copy(
                q_hbm.at[didx.at[b // NCS, b % NCS]], qB, semB)
            gB2 = pltpu.async_copy(
                k_hbm.at[sidx.at[b // NCS, b % NCS]], kB, semB)
            gA1.wait()
            gA2.wait()
            mvec, stA = compute(qA, kA, a, mvec, lbufA, semOA)
            gB1.wait()
            gB2.wait()
            mvec, stB = compute(qB, kB, b, mvec, lbufB, semOB)
            stA.wait()
            stB.wait()
            return mvec

        mvec = lax.fori_loop(
            0, NCH // 2, chunk_pair, jnp.full((16,), -1e30, jnp.float32))
        j = NCH - 1
        pltpu.async_copy(q_hbm.at[didx.at[NSL - 1, NCS - 1]], qA, semA).wait()
        pltpu.async_copy(k_hbm.at[sidx.at[NSL - 1, NCS - 1]], kA, semA).wait()
        mvec, stT = compute(qA, kA, j, mvec, lbufA, semOA)
        stT.wait()
        mbuf[...] = mvec
        pltpu.sync_copy(mbuf, mx_out.at[pl.ds(w * 16, 16)])

    return k(q, kk, src_rs, dst_rs)


def _sc_agg(v, logits, maxes, src_rs, dst_rs, zeros128):
    @functools.partial(
        pl.kernel,
        out_type=(jax.ShapeDtypeStruct((NC, NP, H), jnp.float32),
                  jax.ShapeDtypeStruct((NC, NP, H), jnp.float32)),
        mesh=_mesh(),
        scratch_types=[
            pltpu.VMEM((NCS, C), jnp.int32),
            pltpu.VMEM((NCS, C), jnp.int32),
            pltpu.VMEM((C,), jnp.float32),
            pltpu.VMEM((C,), jnp.float32),
            pltpu.VMEM((C, H), jnp.float32),
            pltpu.VMEM((C, H), jnp.float32),
            pltpu.VMEM((NW * 16,), jnp.float32),
            pltpu.VMEM_SHARED((NP, H), jnp.float32),
            pltpu.SemaphoreType.DMA,
            pltpu.SemaphoreType.DMA,
            pltpu.SemaphoreType.DMA,
            pltpu.SemaphoreType.DMA,
            pltpu.SemaphoreType.DMA,
            pltpu.SemaphoreType.DMA,
        ],
    )
    def k(v_hbm, lg_hbm, mx_hbm, src_hbm, dst_hbm, z_hbm,
          num_out, den_out,
          sidx, didx, lgcA, lgcB, scbufA, scbufB, mx_v, acc,
          semA, semB, semLA, semLB, semSA, semSB):
        c = lax.axis_index("c")
        s = lax.axis_index("s")
        w = _wid()
        pltpu.sync_copy(z_hbm, acc.at[pl.ds(s * RPS, RPS)])
        pltpu.sync_copy(mx_hbm, mx_v)
        m = mx_v[pl.ds(0, 16)]
        for i in range(1, NW):
            m = jnp.maximum(m, mx_v[pl.ds(i * 16, 16)])
        g0 = jnp.maximum(jnp.maximum(m[0], m[1]), jnp.maximum(m[2], m[3]))
        g1 = jnp.maximum(jnp.maximum(m[4], m[5]), jnp.maximum(m[6], m[7]))
        g2 = jnp.maximum(jnp.maximum(m[8], m[9]), jnp.maximum(m[10], m[11]))
        g3 = jnp.maximum(jnp.maximum(m[12], m[13]), jnp.maximum(m[14], m[15]))
        gmax = jnp.maximum(jnp.maximum(g0, g1), jnp.maximum(g2, g3))
        iota = lax.iota(jnp.int32, 16)
        plsc.subcore_barrier()

        def scale(scbuf, lgc):
            def group(g, cy):
                lv = lgc[pl.ds(g * 16, 16)]
                exl = jnp.exp(lv - gmax)
                for l in range(16):
                    i = g * 16 + l
                    ce = exl[l]
                    row_v = scbuf.at[i]
                    for b in range(8):
                        row_v[pl.ds(b * 16, 16)] = (
                            row_v[pl.ds(b * 16, 16)] * ce)
                return cy

            lax.fori_loop(0, G16, group, 0)

        def slab(mm, carry):
            pltpu.sync_copy(src_hbm.at[w, mm], sidx)
            pltpu.sync_copy(dst_hbm.at[w, mm], didx)
            base = w * EPW + mm * NCS * C

            def pair(kk, cy):
                a = 2 * kk
                b = 2 * kk + 1
                gA = pltpu.async_copy(v_hbm.at[sidx.at[a]], scbufA, semA)
                lA = pltpu.async_copy(
                    lg_hbm.at[pl.ds(base + a * C, C)], lgcA, semLA)
                gB = pltpu.async_copy(v_hbm.at[sidx.at[b]], scbufB, semB)
                lB = pltpu.async_copy(
                    lg_hbm.at[pl.ds(base + b * C, C)], lgcB, semLB)
                gA.wait()
                lA.wait()
                scale(scbufA, lgcA)
                sA = pltpu.async_copy(scbufA, acc.at[didx.at[a]], semSA,
                                      add=True)
                gB.wait()
                lB.wait()
                scale(scbufB, lgcB)
                sB = pltpu.async_copy(scbufB, acc.at[didx.at[b]], semSB,
                                      add=True)
                sA.wait()
                sB.wait()
                return cy

            lax.fori_loop(0, NCS // 2, pair, 0)
            a = NCS - 1
            pltpu.async_copy(v_hbm.at[sidx.at[a]], scbufA, semA).wait()
            pltpu.sync_copy(lg_hbm.at[pl.ds(base + a * C, C)], lgcA)
            scale(scbufA, lgcA)
            pltpu.sync_copy(scbufA, acc.at[didx.at[a]], add=True)
            return carry

        lax.fori_loop(0, NSL, slab, 0)
        plsc.subcore_barrier()
        pltpu.sync_copy(acc.at[pl.ds(s * RPS, RPS)],
                        num_out.at[c, pl.ds(s * RPS, RPS)])
        plsc.subcore_barrier()
        pltpu.sync_copy(z_hbm, acc.at[pl.ds(s * RPS, RPS)])
        plsc.subcore_barrier()

        zv = jnp.zeros((16,), jnp.float32)

        def zrow(r, carry):
            rvA = scbufA.at[r]
            rvB = scbufB.at[r]
            for b in range(8):
                rvA[pl.ds(b * 16, 16)] = zv
                rvB[pl.ds(b * 16, 16)] = zv
            return carry

        lax.fori_loop(0, C, zrow, 0)

        def onehot(scbuf, lgc):
            def group(g, cy):
                lv = lgc[pl.ds(g * 16, 16)]
                exl = jnp.exp(lv - gmax)
                for l in range(16):
                    i = g * 16 + l
                    scbuf.at[i][pl.ds(0, 16)] = jnp.where(
                        iota == 0, exl[l], 0.0)
                return cy

            lax.fori_loop(0, G16, group, 0)

        def slab2(mm, carry):
            pltpu.sync_copy(dst_hbm.at[w, mm], didx)
            base = w * EPW + mm * NCS * C

            def pair(kk, cy):
                a = 2 * kk
                b = 2 * kk + 1
                lA = pltpu.async_copy(
                    lg_hbm.at[pl.ds(base + a * C, C)], lgcA, semLA)
                lB = pltpu.async_copy(
                    lg_hbm.at[pl.ds(base + b * C, C)], lgcB, semLB)
                lA.wait()
                onehot(scbufA, lgcA)
                sA = pltpu.async_copy(scbufA, acc.at[didx.at[a]], semSA,
                                      add=True)
                lB.wait()
                onehot(scbufB, lgcB)
                sB = pltpu.async_copy(scbufB, acc.at[didx.at[b]], semSB,
                                      add=True)
                sA.wait()
                sB.wait()
                return cy

            lax.fori_loop(0, NCS // 2, pair, 0)
            a = NCS - 1
            pltpu.sync_copy(lg_hbm.at[pl.ds(base + a * C, C)], lgcA)
            onehot(scbufA, lgcA)
            pltpu.sync_copy(scbufA, acc.at[didx.at[a]], add=True)
            return carry

        lax.fori_loop(0, NSL, slab2, 0)
        plsc.subcore_barrier()
        pltpu.sync_copy(acc.at[pl.ds(s * RPS, RPS)],
                        den_out.at[c, pl.ds(s * RPS, RPS)])

    return k(v, logits, maxes, src_rs, dst_rs, zeros128)


def _row_spec(cols):
    return pl.BlockSpec((RB, cols), lambda i: (i, 0))


def _full_spec(shape):
    nd = len(shape)
    if nd == 2:
        return pl.BlockSpec(shape, lambda i: (0, 0))
    return pl.BlockSpec(shape, lambda i: (0,) * nd)


def _stack_spec(cols):
    return pl.BlockSpec((NC, RB, cols), lambda i: (0, i, 0))


def _tc1(x, Wgc1, deg2):
    def body(x_ref, w_ref, deg_ref, dinv_ref, hs_ref):
        d = deg_ref[0][:, :1] + deg_ref[1][:, :1] + 1.0
        dinv = lax.rsqrt(d)
        h = jnp.dot(x_ref[...], w_ref[...],
                    preferred_element_type=jnp.float32)
        dinv_ref[...] = dinv
        hs_ref[...] = h * dinv

    return pl.pallas_call(
        body,
        grid=(NP // RB,),
        in_specs=[_row_spec(D), _full_spec((D, H)), _stack_spec(H)],
        out_specs=[_row_spec(1), _row_spec(H)],
        out_shape=[jax.ShapeDtypeStruct((NP, 1), jnp.float32),
                   jax.ShapeDtypeStruct((NP, H), jnp.float32)],
    )(x, Wgc1, deg2)


def _tc2(agg1, hs1, dinv, Wgc2, bgc1):
    def body(agg_ref, hs_ref, dinv_ref, w_ref, b_ref, out_ref):
        dinv = dinv_ref[...]
        x2 = jax.nn.relu(dinv * (agg_ref[0] + agg_ref[1] + hs_ref[...])
                         + b_ref[...])
        h2 = jnp.dot(x2, w_ref[...], preferred_element_type=jnp.float32)
        out_ref[...] = h2 * dinv

    return pl.pallas_call(
        body,
        grid=(NP // RB,),
        in_specs=[_stack_spec(H), _row_spec(H), _row_spec(1),
                  _full_spec((H, H)), _full_spec((1, H))],
        out_specs=_row_spec(H),
        out_shape=jax.ShapeDtypeStruct((NP, H), jnp.float32),
    )(agg1, hs1, dinv, Wgc2, bgc1)


def _tc3(agg2, hs2, dinv, bgc2, x, protos, Wq_a, bq_a, Wk_a, bk_a, Wv_a,
         bv_a, Wo_a, bo_a, Wf, bf, Wq1, bq1, Wk1, bk1, Wv1, bv1, Ws1, bs1):
    scale = 1.0 / math.sqrt(float(H))

    def body(agg_ref, hs_ref, dinv_ref, bgc2_ref, x_ref, pr_ref,
             wqa_ref, bqa_ref, wka_ref, bka_ref, wva_ref, bva_ref,
             woa_ref, boa_ref, wf_ref, bf_ref,
             wq1_ref, bq1_ref, wk1_ref, bk1_ref, wv1_ref, bv1_ref,
             ws1_ref, bs1_ref,
             q1_ref, k1_ref, v1_ref, s1_ref):
        dinv = dinv_ref[...]
        x3 = jax.nn.relu(dinv * (agg_ref[0] + agg_ref[1] + hs_ref[...])
                         + bgc2_ref[...])
        qa = jnp.dot(x3, wqa_ref[...],
                     preferred_element_type=jnp.float32) + bqa_ref[...]
        pr = pr_ref[...]
        ka = jnp.dot(pr, wka_ref[...],
                     preferred_element_type=jnp.float32) + bka_ref[...]
        va = jnp.dot(pr, wva_ref[...],
                     preferred_element_type=jnp.float32) + bva_ref[...]
        lg = lax.dot_general(qa, ka, (((1,), (1,)), ((), ())),
                             preferred_element_type=jnp.float32) * scale
        lg = lg - jnp.max(lg, axis=-1, keepdims=True)
        ex = jnp.exp(lg)
        attn = ex / jnp.sum(ex, axis=-1, keepdims=True)
        nr = jnp.dot(jnp.dot(attn, va, preferred_element_type=jnp.float32),
                     woa_ref[...],
                     preferred_element_type=jnp.float32) + boa_ref[...]
        ftb = jnp.dot(x_ref[...], wf_ref[...],
                      preferred_element_type=jnp.float32) + bf_ref[...]
        z = jnp.concatenate([ftb, nr], axis=-1)
        q1_ref[...] = jnp.dot(z, wq1_ref[...],
                              preferred_element_type=jnp.float32) + bq1_ref[...]
        k1_ref[...] = jnp.dot(z, wk1_ref[...],
                              preferred_element_type=jnp.float32) + bk1_ref[...]
        v1_ref[...] = jnp.dot(z, wv1_ref[...],
                              preferred_element_type=jnp.float32) + bv1_ref[...]
        s1_ref[...] = jnp.dot(z, ws1_ref[...],
                              preferred_element_type=jnp.float32) + bs1_ref[...]

    z_in = FT + H
    return pl.pallas_call(
        body,
        grid=(NP // RB,),
        in_specs=[_stack_spec(H), _row_spec(H), _row_spec(1),
                  _full_spec((1, H)), _row_spec(D), _full_spec((P, H)),
                  _full_spec((H, H)), _full_spec((1, H)),
                  _full_spec((H, H)), _full_spec((1, H)),
                  _full_spec((H, H)), _full_spec((1, H)),
                  _full_spec((H, H)), _full_spec((1, H)),
                  _full_spec((D, FT)), _full_spec((1, FT)),
                  _full_spec((z_in, H)), _full_spec((1, H)),
                  _full_spec((z_in, H)), _full_spec((1, H)),
                  _full_spec((z_in, H)), _full_spec((1, H)),
                  _full_spec((z_in, H)), _full_spec((1, H))],
        out_specs=[_row_spec(H)] * 4,
        out_shape=[jax.ShapeDtypeStruct((NP, H), jnp.float32)] * 4,
    )(agg2, hs2, dinv, bgc2, x, protos, Wq_a, bq_a, Wk_a, bk_a, Wv_a, bv_a,
      Wo_a, bo_a, Wf, bf, Wq1, bq1, Wk1, bk1, Wv1, bv1, Ws1, bs1)


def _ln_block(z, g, b):
    mu = jnp.mean(z, axis=-1, keepdims=True)
    var = jnp.mean((z - mu) ** 2, axis=-1, keepdims=True)
    return g * (z - mu) / jnp.sqrt(var + 1e-5) + b


def _tc4(num1, den1, skip1, g1, be1, Wq2, bq2, Wk2, bk2, Wv2, bv2, Ws2, bs2):
    def body(num_ref, den_ref, sk_ref, g_ref, b_ref,
             wq_ref, bq_ref, wk_ref, bk_ref, wv_ref, bv_ref,
             ws_ref, bs_ref, q_ref, k_ref, v_ref, s_ref):
        num = num_ref[0] + num_ref[1]
        den = den_ref[0][:, :1] + den_ref[1][:, :1]
        z = num / (den + 1e-16) + sk_ref[...]
        z = _ln_block(z, g_ref[...], b_ref[...])
        q_ref[...] = jnp.dot(z, wq_ref[...],
                             preferred_element_type=jnp.float32) + bq_ref[...]
        k_ref[...] = jnp.dot(z, wk_ref[...],
                             preferred_element_type=jnp.float32) + bk_ref[...]
        v_ref[...] = jnp.dot(z, wv_ref[...],
                             preferred_element_type=jnp.float32) + bv_ref[...]
        s_ref[...] = jnp.dot(z, ws_ref[...],
                             preferred_element_type=jnp.float32) + bs_ref[...]

    return pl.pallas_call(
        body,
        grid=(NP // RB,),
        in_specs=[_stack_spec(H), _stack_spec(H), _row_spec(H),
                  _full_spec((1, H)), _full_spec((1, H)),
                  _full_spec((H, H)), _full_spec((1, H)),
                  _full_spec((H, H)), _full_spec((1, H)),
                  _full_spec((H, H)), _full_spec((1, H)),
                  _full_spec((H, H)), _full_spec((1, H))],
        out_specs=[_row_spec(H)] * 4,
        out_shape=[jax.ShapeDtypeStruct((NP, H), jnp.float32)] * 4,
    )(num1, den1, skip1, g1, be1, Wq2, bq2, Wk2, bk2, Wv2, bv2, Ws2, bs2)


def _tc5(num2, den2, skip2, g2, be2, Wm0, bm0, Wm1, bm1):
    def body(num_ref, den_ref, sk_ref, g_ref, b_ref, wm0_ref, bm0_ref,
             wm1_ref, bm1_ref, out_ref):
        num = num_ref[0] + num_ref[1]
        den = den_ref[0][:, :1] + den_ref[1][:, :1]
        z = num / (den + 1e-16) + sk_ref[...]
        z = _ln_block(z, g_ref[...], b_ref[...])
        y = jnp.dot(z, wm0_ref[...],
                    preferred_element_type=jnp.float32) + bm0_ref[...]
        y = 0.5 * y * (1.0 + lax.erf(y / math.sqrt(2.0)))
        y = jnp.dot(y, wm1_ref[...],
                    preferred_element_type=jnp.float32) + bm1_ref[...]
        out_ref[...] = jax.nn.sigmoid(y)

    return pl.pallas_call(
        body,
        grid=(NP // RB,),
        in_specs=[_stack_spec(H), _stack_spec(H), _row_spec(H),
                  _full_spec((1, H)), _full_spec((1, H)),
                  _full_spec((H, H // 2)), _full_spec((1, H // 2)),
                  _full_spec((H // 2, 1)), _full_spec((1, 1))],
        out_specs=_row_spec(1),
        out_shape=jax.ShapeDtypeStruct((NP, 1), jnp.float32),
    )(num2, den2, skip2, g2, be2, Wm0, bm0, Wm1, bm1)


def kernel(x, edge_index, prototypes, Wgc1, bgc1, Wgc2, bgc2, Wq_a, bq_a,
           Wk_a, bk_a, Wv_a, bv_a, Wo_a, bo_a, Wf, bf, Wq1, bq1, Wk1, bk1,
           Wv1, bv1, Ws1, bs1, g1, be1, Wq2, bq2, Wk2, bk2, Wv2, bv2, Ws2,
           bs2, g2, be2, Wm0, bm0, Wm1, bm1):
    src_rs = edge_index[0].reshape(NW, NSL, NCS, C)
    dst_rs = edge_index[1].reshape(NW, NSL, NCS, C)
    x = jnp.pad(x, ((0, NP - N), (0, 0)))
    onerows = jnp.tile(
        (lax.iota(jnp.int32, H) == 0).astype(jnp.float32)[None, :], (C, 1))
    zeros128 = jnp.zeros((RPS, H), jnp.float32)

    r2 = lambda b: b.reshape(1, -1)

    deg2 = _sc_deg(dst_rs, onerows, zeros128)
    dinv, hs1 = _tc1(x, Wgc1, deg2)
    agg1 = _sc_gcn(hs1, src_rs, dst_rs, zeros128)
    hs2 = _tc2(agg1, hs1, dinv, Wgc2, r2(bgc1))
    agg2 = _sc_gcn(hs2, src_rs, dst_rs, zeros128)
    q1, k1, v1, s1 = _tc3(
        agg2, hs2, dinv, r2(bgc2), x, prototypes,
        Wq_a, r2(bq_a), Wk_a, r2(bk_a), Wv_a, r2(bv_a), Wo_a, r2(bo_a),
        Wf, r2(bf), Wq1, r2(bq1), Wk1, r2(bk1), Wv1, r2(bv1), Ws1, r2(bs1))
    lg1, mx1 = _sc_logits(q1, k1, src_rs, dst_rs)
    num1, den1 = _sc_agg(v1, lg1, mx1, src_rs, dst_rs, zeros128)
    q2, k2, v2, s2 = _tc4(
        num1, den1, s1, r2(g1), r2(be1),
        Wq2, r2(bq2), Wk2, r2(bk2), Wv2, r2(bv2), Ws2, r2(bs2))
    lg2, mx2 = _sc_logits(q2, k2, src_rs, dst_rs)
    num2, den2 = _sc_agg(v2, lg2, mx2, src_rs, dst_rs, zeros128)
    out = _tc5(num2, den2, s2, r2(g2), r2(be2), Wm0, r2(bm0), Wm1, r2(bm1))
    return out[:N]
